# baseline (speedup 1.0000x reference)
import functools

import jax
import jax.numpy as jnp
from jax import lax
from jax.experimental import pallas as pl
from jax.experimental.pallas import tpu as pltpu

K = 16
NEG = None


def _tc_knn_body(pts_ref, ptsn_ref, dist_ref, idx_ref, xout_ref, xq_ref):
    t = pl.program_id(1)

    @pl.when(t == 0)
    def _():
        xq_ref[...] = pts_ref[0, 0]

    xkt = ptsn_ref[0, 0]
    xq = xq_ref[...]
    n = xq.shape[1]
    kk = jnp.sum(xkt * xkt, axis=1, keepdims=True)
    qq = jnp.sum(xq * xq, axis=0)
    xkb = xkt.astype(jnp.bfloat16).astype(jnp.float32)
    xqb = xq.astype(jnp.bfloat16).astype(jnp.float32)
    dot = (xkb[:, 0:1] * xqb[0][None, :]
           + xkb[:, 1:2] * xqb[1][None, :]
           + xkb[:, 2:3] * xqb[2][None, :])
    d = (qq[None, :] + kk) - 2.0 * dot

    iota_k = lax.broadcasted_iota(jnp.int32, (n, n), 0)
    inf = jnp.float32(jnp.inf)
    zero = jnp.float32(0.0)
    m_prev = None
    i_prev = None
    for j in range(K):
        if j == 0:
            dv = d
        else:
            valid = ((d > m_prev[None, :])
                     | ((d == m_prev[None, :]) & (iota_k > i_prev[None, :])))
            dv = jnp.where(valid, d, inf)
        m = jnp.min(dv, axis=0)
        cand = jnp.where(dv == m[None, :], iota_k, n)
        aidx = jnp.min(cand, axis=0)
        dist_ref[0, 0, j, :] = m
        idx_ref[0, 0, j, :] = aidx
        if j == 0:
            first = iota_k == aidx[None, :]
            for c in range(3):
                xnew_c = jnp.sum(jnp.where(first, xkt[:, c:c + 1], zero), axis=0)
                xq_ref[c, :] = xnew_c
                xout_ref[0, 0, c, :] = xnew_c
        m_prev, i_prev = m, aidx


def _tc_knn(pts8, ptsn8):
    b, t, c8, n = pts8.shape
    grid = (b, t)
    return pl.pallas_call(
        _tc_knn_body,
        grid=grid,
        in_specs=[
            pl.BlockSpec((1, 1, c8, n), lambda bi, ti: (bi, jnp.maximum(ti - 1, 0), 0, 0)),
            pl.BlockSpec((1, 1, n, c8), lambda bi, ti: (bi, jnp.maximum(ti - 1, 0), 0, 0)),
        ],
        out_specs=[
            pl.BlockSpec((1, 1, K, n), lambda bi, ti: (bi, ti, 0, 0)),
            pl.BlockSpec((1, 1, K, n), lambda bi, ti: (bi, ti, 0, 0)),
            pl.BlockSpec((1, 1, c8, n), lambda bi, ti: (bi, ti, 0, 0)),
        ],
        out_shape=[
            jax.ShapeDtypeStruct((b, t, K, n), jnp.float32),
            jax.ShapeDtypeStruct((b, t, K, n), jnp.int32),
            jax.ShapeDtypeStruct((b, t, c8, n), jnp.float32),
        ],
        scratch_shapes=[pltpu.VMEM((c8, n), jnp.float32)],
        compiler_params=pltpu.CompilerParams(
            dimension_semantics=("arbitrary", "arbitrary"),
        ),
    )(pts8, ptsn8)


def kernel(point_seq):
    b, t, n, d = point_seq.shape
    ptsT = jnp.transpose(point_seq, (0, 1, 3, 2))
    pts8 = jnp.concatenate(
        [ptsT, jnp.zeros((b, t, 8 - d, n), jnp.float32)], axis=2)
    ptsn8 = jnp.concatenate(
        [point_seq, jnp.zeros((b, t, n, 8 - d), jnp.float32)], axis=3)

    dist_kn, idx_kn, xout8 = _tc_knn(pts8, ptsn8)

    x_out = jnp.transpose(xout8[:, :, :d, :], (0, 1, 3, 2))

    idxs = jnp.transpose(idx_kn, (0, 1, 3, 2))
    distances = jnp.transpose(dist_kn, (0, 1, 3, 2))
    x2 = jnp.concatenate([point_seq[:, :1], point_seq], axis=1)[:, :-1]
    gathered = jax.vmap(jax.vmap(lambda p, i: p[i]))(x2, idxs)
    anchor = x_out[:, 0][:, None, :, None, :]
    normalized = gathered - anchor
    patchlet_feats = jnp.concatenate([gathered, normalized], axis=-1)
    return patchlet_feats, gathered, distances, idxs, x_out


# speedup vs baseline: 12.6754x; 12.6754x over previous
import functools

import jax
import jax.numpy as jnp
from jax import lax
from jax.experimental import pallas as pl
from jax.experimental.pallas import tpu as pltpu
from jax.experimental.pallas import tpu_sc as plsc

K = 16


def _tc_knn_body(pts_ref, ptsn_ref, dist_ref, idx_ref, xout_ref, xq_ref):
    t = pl.program_id(1)

    @pl.when(t == 0)
    def _():
        xq_ref[...] = pts_ref[0, 0]

    xkt = ptsn_ref[0, 0]
    xq = xq_ref[...]
    n = xq.shape[1]
    kk = jnp.sum(xkt * xkt, axis=1, keepdims=True)
    qq = jnp.sum(xq * xq, axis=0)
    xkb = xkt.astype(jnp.bfloat16).astype(jnp.float32)
    xqb = xq.astype(jnp.bfloat16).astype(jnp.float32)
    dot = (xkb[:, 0:1] * xqb[0][None, :]
           + xkb[:, 1:2] * xqb[1][None, :]
           + xkb[:, 2:3] * xqb[2][None, :])
    d = (qq[None, :] + kk) - 2.0 * dot

    iota_k = lax.broadcasted_iota(jnp.int32, (n, n), 0)
    inf = jnp.float32(jnp.inf)
    zero = jnp.float32(0.0)
    for j in range(K):
        m = jnp.min(d, axis=0)
        cand = jnp.where(d == m[None, :], iota_k, n)
        aidx = jnp.min(cand, axis=0)
        first = cand == aidx[None, :]
        dist_ref[0, 0, j, :] = m
        idx_ref[0, 0, j, :] = aidx
        if j == 0:
            for c in range(3):
                xnew_c = jnp.sum(jnp.where(first, xkt[:, c:c + 1], zero), axis=0)
                xq_ref[c, :] = xnew_c
                xout_ref[0, 0, c, :] = xnew_c
        if j < K - 1:
            d = jnp.where(first, inf, d)


def _tc_knn(pts8, ptsn8):
    b, t, c8, n = pts8.shape
    grid = (b, t)
    return pl.pallas_call(
        _tc_knn_body,
        grid=grid,
        in_specs=[
            pl.BlockSpec((1, 1, c8, n), lambda bi, ti: (bi, jnp.maximum(ti - 1, 0), 0, 0)),
            pl.BlockSpec((1, 1, n, c8), lambda bi, ti: (bi, jnp.maximum(ti - 1, 0), 0, 0)),
        ],
        out_specs=[
            pl.BlockSpec((1, 1, K, n), lambda bi, ti: (bi, ti, 0, 0)),
            pl.BlockSpec((1, 1, K, n), lambda bi, ti: (bi, ti, 0, 0)),
            pl.BlockSpec((1, 1, c8, n), lambda bi, ti: (bi, ti, 0, 0)),
        ],
        out_shape=[
            jax.ShapeDtypeStruct((b, t, K, n), jnp.float32),
            jax.ShapeDtypeStruct((b, t, K, n), jnp.int32),
            jax.ShapeDtypeStruct((b, t, c8, n), jnp.float32),
        ],
        scratch_shapes=[pltpu.VMEM((c8, n), jnp.float32)],
        compiler_params=pltpu.CompilerParams(
            dimension_semantics=("arbitrary", "arbitrary"),
        ),
    )(pts8, ptsn8)


def _sc_gather(pts_flat, idx_kn, dist_kn, anch_flat, b, t, n):
    info = plsc.get_sparse_core_info()
    nc, ns = info.num_cores, info.num_subcores
    nw = nc * ns
    pairs = b * t
    ppw = pairs // nw
    assert pairs % nw == 0 and t % ppw == 0
    qch = 256
    mesh = plsc.VectorSubcoreMesh(core_axis_name="c", subcore_axis_name="s")

    @functools.partial(
        pl.kernel,
        out_type=[
            jax.ShapeDtypeStruct((b, t, n * K * 6), jnp.float32),
            jax.ShapeDtypeStruct((b, t, n * K * 3), jnp.float32),
            jax.ShapeDtypeStruct((b, t, n * K), jnp.float32),
            jax.ShapeDtypeStruct((b, t, n * K), jnp.int32),
        ],
        mesh=mesh,
        scratch_types=[
            pltpu.VMEM((3 * n,), jnp.float32),
            pltpu.VMEM((3 * n,), jnp.float32),
            pltpu.VMEM((K * n,), jnp.int32),
            pltpu.VMEM((K * n,), jnp.float32),
            pltpu.VMEM((qch * K * 6,), jnp.float32),
            pltpu.VMEM((qch * K * 3,), jnp.float32),
            pltpu.VMEM((qch * K,), jnp.float32),
            pltpu.VMEM((qch * K,), jnp.int32),
        ],
        compiler_params=pltpu.CompilerParams(needs_layout_passes=False),
    )
    def sc_fn(pts_hbm, idx_hbm, dist_hbm, anch_hbm,
              feats_hbm, pout_hbm, dout_hbm, iout_hbm,
              tab_v, anch_v, idx_v, dist_v, of_v, op_v, od_v, oi_v):
        cid = lax.axis_index("c")
        sid = lax.axis_index("s")
        wid = sid * nc + cid
        bi = wid // (t // ppw)
        t0 = (wid % (t // ppw)) * ppw
        pltpu.sync_copy(anch_hbm.at[bi], anch_v)
        iota = jnp.arange(K, dtype=jnp.int32)
        i3 = iota * 3
        i6 = iota * 6
        iN = iota * n
        for p in range(ppw):
            ti = t0 + p
            tbl = jnp.maximum(ti - 1, 0)
            pltpu.sync_copy(pts_hbm.at[bi, tbl], tab_v)
            pltpu.sync_copy(idx_hbm.at[bi, ti], idx_v)
            pltpu.sync_copy(dist_hbm.at[bi, ti], dist_v)
            for ch in range(n // qch):

                def qbody(qi, carry):
                    qg = ch * qch + qi
                    qv = jnp.full((K,), qg, dtype=jnp.int32)
                    nbr = plsc.load_gather(idx_v, [iN + qg])
                    dd = plsc.load_gather(dist_v, [iN + qg])
                    oi_v[pl.ds(qi * K, K)] = nbr
                    od_v[pl.ds(qi * K, K)] = dd
                    gx = plsc.load_gather(tab_v, [nbr])
                    gy = plsc.load_gather(tab_v, [nbr + n])
                    gz = plsc.load_gather(tab_v, [nbr + 2 * n])
                    ax = plsc.load_gather(anch_v, [qv])
                    ay = plsc.load_gather(anch_v, [qv + n])
                    az = plsc.load_gather(anch_v, [qv + 2 * n])
                    b3 = qi * (K * 3)
                    plsc.store_scatter(op_v, [b3 + i3], gx)
                    plsc.store_scatter(op_v, [b3 + i3 + 1], gy)
                    plsc.store_scatter(op_v, [b3 + i3 + 2], gz)
                    b6 = qi * (K * 6)
                    plsc.store_scatter(of_v, [b6 + i6], gx)
                    plsc.store_scatter(of_v, [b6 + i6 + 1], gy)
                    plsc.store_scatter(of_v, [b6 + i6 + 2], gz)
                    plsc.store_scatter(of_v, [b6 + i6 + 3], gx - ax)
                    plsc.store_scatter(of_v, [b6 + i6 + 4], gy - ay)
                    plsc.store_scatter(of_v, [b6 + i6 + 5], gz - az)
                    return carry

                lax.fori_loop(0, qch, qbody, 0, unroll=False)
                off = ch * qch * K
                pltpu.sync_copy(of_v, feats_hbm.at[bi, ti, pl.ds(off * 6, qch * K * 6)])
                pltpu.sync_copy(op_v, pout_hbm.at[bi, ti, pl.ds(off * 3, qch * K * 3)])
                pltpu.sync_copy(od_v, dout_hbm.at[bi, ti, pl.ds(off, qch * K)])
                pltpu.sync_copy(oi_v, iout_hbm.at[bi, ti, pl.ds(off, qch * K)])

    return sc_fn(pts_flat, idx_kn, dist_kn, anch_flat)


def kernel(point_seq):
    b, t, n, d = point_seq.shape
    ptsT = jnp.transpose(point_seq, (0, 1, 3, 2))
    pts8 = jnp.concatenate(
        [ptsT, jnp.zeros((b, t, 8 - d, n), jnp.float32)], axis=2)
    ptsn8 = jnp.concatenate(
        [point_seq, jnp.zeros((b, t, n, 8 - d), jnp.float32)], axis=3)

    dist_kn, idx_kn, xout8 = _tc_knn(pts8, ptsn8)

    x_out = jnp.transpose(xout8[:, :, :d, :], (0, 1, 3, 2))

    pts_flat = ptsT.reshape(b, t, 3 * n)
    anch_flat = xout8[:, 0, :3, :].reshape(b, 3 * n)
    feats_f, pts_f, dist_f, idx_f = _sc_gather(
        pts_flat, idx_kn.reshape(b, t, K * n), dist_kn.reshape(b, t, K * n),
        anch_flat, b, t, n)
    patchlet_feats = feats_f.reshape(b, t, n, K, 6)
    patchlet_points = pts_f.reshape(b, t, n, K, 3)
    distances = dist_f.reshape(b, t, n, K)
    idxs = idx_f.reshape(b, t, n, K)
    return patchlet_feats, patchlet_points, distances, idxs, x_out


# speedup vs baseline: 14.2386x; 1.1233x over previous
import functools

import jax
import jax.numpy as jnp
from jax import lax
from jax.experimental import pallas as pl
from jax.experimental.pallas import tpu as pltpu
from jax.experimental.pallas import tpu_sc as plsc

K = 16


def _tc_knn_body(pts_ref, ptsn_ref, dist_ref, idx_ref, xout_ref, xq_ref):
    t = pl.program_id(1)

    @pl.when(t == 0)
    def _():
        xq_ref[...] = pts_ref[0, 0]

    xkt = ptsn_ref[0, 0]
    xq = xq_ref[...]
    n = xq.shape[1]
    kk = jnp.sum(xkt * xkt, axis=1, keepdims=True)
    qq = jnp.sum(xq * xq, axis=0)
    xkb = xkt.astype(jnp.bfloat16).astype(jnp.float32)
    xqb = xq.astype(jnp.bfloat16).astype(jnp.float32)
    dot = (xkb[:, 0:1] * xqb[0][None, :]
           + xkb[:, 1:2] * xqb[1][None, :]
           + xkb[:, 2:3] * xqb[2][None, :])
    d = (qq[None, :] + kk) - 2.0 * dot

    iota_f = lax.broadcasted_iota(jnp.int32, (n, n), 0).astype(jnp.float32)
    inf = jnp.float32(jnp.inf)
    zero = jnp.float32(0.0)
    nf = jnp.float32(n)
    for j in range(K):
        m = jnp.min(d, axis=0)
        cand = jnp.where(d == m[None, :], iota_f, nf)
        aidx = jnp.min(cand, axis=0)
        first = cand == aidx[None, :]
        dist_ref[0, 0, j, :] = m
        idx_ref[0, 0, j, :] = aidx.astype(jnp.int32)
        if j == 0:
            for c in range(3):
                xnew_c = jnp.sum(jnp.where(first, xkt[:, c:c + 1], zero), axis=0)
                xq_ref[c, :] = xnew_c
                xout_ref[0, 0, c, :] = xnew_c
        if j < K - 1:
            d = jnp.where(first, inf, d)


def _tc_knn(ptsc, ptsn):
    b, t, c3, n = ptsc.shape
    grid = (b, t)
    return pl.pallas_call(
        _tc_knn_body,
        grid=grid,
        in_specs=[
            pl.BlockSpec((1, 1, c3, n), lambda bi, ti: (bi, jnp.maximum(ti - 1, 0), 0, 0)),
            pl.BlockSpec((1, 1, n, c3), lambda bi, ti: (bi, jnp.maximum(ti - 1, 0), 0, 0)),
        ],
        out_specs=[
            pl.BlockSpec((1, 1, K, n), lambda bi, ti: (bi, ti, 0, 0)),
            pl.BlockSpec((1, 1, K, n), lambda bi, ti: (bi, ti, 0, 0)),
            pl.BlockSpec((1, 1, c3, n), lambda bi, ti: (bi, ti, 0, 0)),
        ],
        out_shape=[
            jax.ShapeDtypeStruct((b, t, K, n), jnp.float32),
            jax.ShapeDtypeStruct((b, t, K, n), jnp.int32),
            jax.ShapeDtypeStruct((b, t, c3, n), jnp.float32),
        ],
        scratch_shapes=[pltpu.VMEM((c3, n), jnp.float32)],
        compiler_params=pltpu.CompilerParams(
            dimension_semantics=("arbitrary", "arbitrary"),
        ),
    )(ptsc, ptsn)


def _sc_gather(pts_flat, idx_kn, dist_kn, anch_flat, b, t, n):
    info = plsc.get_sparse_core_info()
    nc, ns = info.num_cores, info.num_subcores
    nw = nc * ns
    pairs = b * t
    ppw = pairs // nw
    assert pairs % nw == 0 and t % ppw == 0
    qch = 256
    mesh = plsc.VectorSubcoreMesh(core_axis_name="c", subcore_axis_name="s")

    @functools.partial(
        pl.kernel,
        out_type=[
            jax.ShapeDtypeStruct((b, t, n * K * 6), jnp.float32),
            jax.ShapeDtypeStruct((b, t, n * K * 3), jnp.float32),
            jax.ShapeDtypeStruct((b, t, n * K), jnp.float32),
            jax.ShapeDtypeStruct((b, t, n * K), jnp.int32),
        ],
        mesh=mesh,
        scratch_types=[
            pltpu.VMEM((3 * n,), jnp.float32),
            pltpu.VMEM((3 * n,), jnp.float32),
            pltpu.VMEM((K * n,), jnp.int32),
            pltpu.VMEM((K * n,), jnp.float32),
            pltpu.VMEM((qch * K * 6,), jnp.float32),
            pltpu.VMEM((qch * K * 3,), jnp.float32),
            pltpu.VMEM((qch * K,), jnp.float32),
            pltpu.VMEM((qch * K,), jnp.int32),
        ],
        compiler_params=pltpu.CompilerParams(needs_layout_passes=False),
    )
    def sc_fn(pts_hbm, idx_hbm, dist_hbm, anch_hbm,
              feats_hbm, pout_hbm, dout_hbm, iout_hbm,
              tab_v, anch_v, idx_v, dist_v, of_v, op_v, od_v, oi_v):
        cid = lax.axis_index("c")
        sid = lax.axis_index("s")
        wid = sid * nc + cid
        bi = wid // (t // ppw)
        t0 = (wid % (t // ppw)) * ppw
        pltpu.sync_copy(anch_hbm.at[bi], anch_v)
        iota = jnp.arange(K, dtype=jnp.int32)
        i3 = iota * 3
        i6 = iota * 6
        iN = iota * n
        for p in range(ppw):
            ti = t0 + p
            tbl = jnp.maximum(ti - 1, 0)
            pltpu.sync_copy(pts_hbm.at[bi, tbl], tab_v)
            pltpu.sync_copy(idx_hbm.at[bi, ti], idx_v)
            pltpu.sync_copy(dist_hbm.at[bi, ti], dist_v)
            for ch in range(n // qch):

                def qbody(qi, carry):
                    qg = ch * qch + qi
                    qv = jnp.full((K,), qg, dtype=jnp.int32)
                    nbr = plsc.load_gather(idx_v, [iN + qg])
                    dd = plsc.load_gather(dist_v, [iN + qg])
                    oi_v[pl.ds(qi * K, K)] = nbr
                    od_v[pl.ds(qi * K, K)] = dd
                    gx = plsc.load_gather(tab_v, [nbr])
                    gy = plsc.load_gather(tab_v, [nbr + n])
                    gz = plsc.load_gather(tab_v, [nbr + 2 * n])
                    ax = plsc.load_gather(anch_v, [qv])
                    ay = plsc.load_gather(anch_v, [qv + n])
                    az = plsc.load_gather(anch_v, [qv + 2 * n])
                    b3 = qi * (K * 3)
                    plsc.store_scatter(op_v, [b3 + i3], gx)
                    plsc.store_scatter(op_v, [b3 + i3 + 1], gy)
                    plsc.store_scatter(op_v, [b3 + i3 + 2], gz)
                    b6 = qi * (K * 6)
                    plsc.store_scatter(of_v, [b6 + i6], gx)
                    plsc.store_scatter(of_v, [b6 + i6 + 1], gy)
                    plsc.store_scatter(of_v, [b6 + i6 + 2], gz)
                    plsc.store_scatter(of_v, [b6 + i6 + 3], gx - ax)
                    plsc.store_scatter(of_v, [b6 + i6 + 4], gy - ay)
                    plsc.store_scatter(of_v, [b6 + i6 + 5], gz - az)
                    return carry

                lax.fori_loop(0, qch, qbody, 0, unroll=False)
                off = ch * qch * K
                pltpu.sync_copy(of_v, feats_hbm.at[bi, ti, pl.ds(off * 6, qch * K * 6)])
                pltpu.sync_copy(op_v, pout_hbm.at[bi, ti, pl.ds(off * 3, qch * K * 3)])
                pltpu.sync_copy(od_v, dout_hbm.at[bi, ti, pl.ds(off, qch * K)])
                pltpu.sync_copy(oi_v, iout_hbm.at[bi, ti, pl.ds(off, qch * K)])

    return sc_fn(pts_flat, idx_kn, dist_kn, anch_flat)


def kernel(point_seq):
    b, t, n, d = point_seq.shape
    ptsT = jnp.transpose(point_seq, (0, 1, 3, 2))

    dist_kn, idx_kn, xout3 = _tc_knn(ptsT, point_seq)

    x_out = jnp.transpose(xout3, (0, 1, 3, 2))

    pts_flat = ptsT.reshape(b, t, 3 * n)
    anch_flat = xout3[:, 0].reshape(b, 3 * n)
    feats_f, pts_f, dist_f, idx_f = _sc_gather(
        pts_flat, idx_kn.reshape(b, t, K * n), dist_kn.reshape(b, t, K * n),
        anch_flat, b, t, n)
    patchlet_feats = feats_f.reshape(b, t, n, K, 6)
    patchlet_points = pts_f.reshape(b, t, n, K, 3)
    distances = dist_f.reshape(b, t, n, K)
    idxs = idx_f.reshape(b, t, n, K)
    return patchlet_feats, patchlet_points, distances, idxs, x_out
